# baseline (speedup 1.0000x reference)
import functools

import jax
import jax.numpy as jnp
import numpy as np
from jax.experimental import pallas as pl

_EPS = 1000.0 * float(np.finfo(np.float32).eps)
_TOK = 2048
_CHUNK = 512


def _fused_kernel(coord_ref, x_ref, w_ref, pe_ref, b_ref, o_ref):
    coords = coord_ref[0]
    row = jax.lax.broadcasted_iota(jnp.int32, coords.shape, 0)
    gy = jnp.max(jnp.where(row == 0, coords, -1)) + 1
    gx = jnp.max(jnp.where(row == 1, coords, -1)) + 1
    gyf = gy.astype(jnp.float32)
    gxf = gx.astype(jnp.float32)

    gv = jax.lax.broadcasted_iota(jnp.int32, (48, 256), 0).astype(jnp.float32)
    ki = jax.lax.broadcasted_iota(jnp.int32, (48, 256), 1)
    ka = (ki // 16).astype(jnp.float32)
    kb = (ki % 16).astype(jnp.float32)
    inv_y = 16.0 / gyf
    inv_x = 16.0 / gxf
    ks_y = jnp.maximum(inv_y, 1.0)
    ks_x = jnp.maximum(inv_x, 1.0)
    sy = (gv + 0.5) * inv_y - 0.5
    sx = (gv + 0.5) * inv_x - 0.5
    wy_t = jnp.maximum(0.0, 1.0 - jnp.abs(sy - ka) / ks_y)
    wx_t = jnp.maximum(0.0, 1.0 - jnp.abs(sx - kb) / ks_x)
    ty = jnp.sum(wy_t, axis=1, keepdims=True) * (1.0 / 16.0)
    tx = jnp.sum(wx_t, axis=1, keepdims=True) * (1.0 / 16.0)
    ry = jnp.where(jnp.abs(ty) > _EPS, 1.0 / jnp.where(ty != 0, ty, 1.0), 0.0)
    rx = jnp.where(jnp.abs(tx) > _EPS, 1.0 / jnp.where(tx != 0, tx, 1.0), 0.0)
    wy_t = jnp.where(gv < gyf, wy_t * ry, 0.0)
    wx_t = wx_t * rx

    for tc in range(_TOK // _CHUNK):
        tf = (jax.lax.broadcasted_iota(jnp.int32, (_CHUNK, 1), 0)
              .astype(jnp.float32) + float(tc * _CHUNK))
        g48 = jax.lax.broadcasted_iota(jnp.int32, (_CHUNK, 48), 1).astype(
            jnp.float32)
        u = tf - g48 * gxf
        oy = jnp.where((u >= 0.0) & (u < gxf), 1.0, 0.0)
        xxf = tf - jnp.floor(tf / gxf) * gxf
        ox = jnp.where(g48 == xxf, 1.0, 0.0)

        py = jnp.dot(oy, wy_t, preferred_element_type=jnp.float32)
        px = jnp.dot(ox, wx_t, preferred_element_type=jnp.float32)
        ppe = py * px

        sl = pl.ds(tc * _CHUNK, _CHUNK)
        acc = jax.lax.dot_general(
            x_ref[0, sl, :], w_ref[...], (((1,), (1,)), ((), ())),
            preferred_element_type=jnp.float32)
        acc += jnp.dot(ppe, pe_ref[...], preferred_element_type=jnp.float32)
        o_ref[0, sl, :] = acc + b_ref[...]


@jax.jit
def kernel(patches, patch_coord, pos_embed, proj_w, proj_b):
    n, s, d = patches.shape
    c = proj_w.shape[0]
    coords = jnp.swapaxes(patch_coord, 1, 2)
    w = proj_w
    pe = pos_embed.reshape(256, c)
    b = proj_b.reshape(1, c)
    grid = (n,)
    return pl.pallas_call(
        _fused_kernel,
        grid=grid,
        in_specs=[
            pl.BlockSpec((1, 2, s), lambda i: (i, 0, 0)),
            pl.BlockSpec((1, _TOK, d), lambda i: (i, 0, 0)),
            pl.BlockSpec((c, d), lambda i: (0, 0)),
            pl.BlockSpec((256, c), lambda i: (0, 0)),
            pl.BlockSpec((1, c), lambda i: (0, 0)),
        ],
        out_specs=pl.BlockSpec((1, _TOK, c), lambda i: (i, 0, 0)),
        out_shape=jax.ShapeDtypeStruct((n, s, c), jnp.float32),
    )(coords, patches, w, pe, b)


# speedup vs baseline: 1.0127x; 1.0127x over previous
import functools

import jax
import jax.numpy as jnp
import numpy as np
from jax.experimental import pallas as pl

_EPS = 1000.0 * float(np.finfo(np.float32).eps)
_TOK = 2048
_NB = 384


def _fused_kernel(coord_ref, x_ref, w_ref, pe_ref, b_ref, o_ref):
    coords = coord_ref[0]
    row = jax.lax.broadcasted_iota(jnp.int32, coords.shape, 0)
    gy = jnp.max(jnp.where(row == 0, coords, -1)) + 1
    gx = jnp.max(jnp.where(row == 1, coords, -1)) + 1
    gyf = gy.astype(jnp.float32)
    gxf = gx.astype(jnp.float32)

    gv = jax.lax.broadcasted_iota(jnp.int32, (48, 256), 0).astype(jnp.float32)
    ki = jax.lax.broadcasted_iota(jnp.int32, (48, 256), 1)
    ka = (ki // 16).astype(jnp.float32)
    kb = (ki % 16).astype(jnp.float32)
    inv_y = 16.0 / gyf
    inv_x = 16.0 / gxf
    ks_y = jnp.maximum(inv_y, 1.0)
    ks_x = jnp.maximum(inv_x, 1.0)
    sy = (gv + 0.5) * inv_y - 0.5
    sx = (gv + 0.5) * inv_x - 0.5
    wy_t = jnp.maximum(0.0, 1.0 - jnp.abs(sy - ka) / ks_y)
    wx_t = jnp.maximum(0.0, 1.0 - jnp.abs(sx - kb) / ks_x)
    ty = jnp.sum(wy_t, axis=1, keepdims=True) * (1.0 / 16.0)
    tx = jnp.sum(wx_t, axis=1, keepdims=True) * (1.0 / 16.0)
    ry = jnp.where(jnp.abs(ty) > _EPS, 1.0 / jnp.where(ty != 0, ty, 1.0), 0.0)
    rx = jnp.where(jnp.abs(tx) > _EPS, 1.0 / jnp.where(tx != 0, tx, 1.0), 0.0)
    wy_t = jnp.where(gv < gyf, wy_t * ry, 0.0)
    wx_t = wx_t * rx

    tf = jax.lax.broadcasted_iota(jnp.int32, (_TOK, 1), 0).astype(jnp.float32)
    g48 = jax.lax.broadcasted_iota(jnp.int32, (_TOK, 48), 1).astype(jnp.float32)
    u = tf - g48 * gxf
    oy = jnp.where((u >= 0.0) & (u < gxf), 1.0, 0.0)
    xxf = tf - jnp.floor(tf / gxf) * gxf
    ox = jnp.where(g48 == xxf, 1.0, 0.0)

    py = jnp.dot(oy, wy_t, preferred_element_type=jnp.float32)
    px = jnp.dot(ox, wx_t, preferred_element_type=jnp.float32)
    ppe = py * px

    acc = jax.lax.dot_general(
        x_ref[0], w_ref[...], (((1,), (1,)), ((), ())),
        preferred_element_type=jnp.float32)
    acc += jnp.dot(ppe, pe_ref[...], preferred_element_type=jnp.float32)
    o_ref[0] = acc + b_ref[...]


@jax.jit
def kernel(patches, patch_coord, pos_embed, proj_w, proj_b):
    n, s, d = patches.shape
    c = proj_w.shape[0]
    coords = jnp.swapaxes(patch_coord, 1, 2)
    w = proj_w
    pe = pos_embed.reshape(256, c)
    b = proj_b.reshape(1, c)
    grid = (n,)
    return pl.pallas_call(
        _fused_kernel,
        grid=grid,
        in_specs=[
            pl.BlockSpec((1, 2, s), lambda i: (i, 0, 0)),
            pl.BlockSpec((1, _TOK, d), lambda i: (i, 0, 0)),
            pl.BlockSpec((c, d), lambda i: (0, 0)),
            pl.BlockSpec((256, c), lambda i: (0, 0)),
            pl.BlockSpec((1, c), lambda i: (0, 0)),
        ],
        out_specs=pl.BlockSpec((1, _TOK, c), lambda i: (i, 0, 0)),
        out_shape=jax.ShapeDtypeStruct((n, s, c), jnp.float32),
    )(coords, patches, w, pe, b)
